# baseline (speedup 1.0000x reference)
import jax
import jax.numpy as jnp
import numpy as np
from jax import lax
from jax.experimental import pallas as pl
from jax.experimental.pallas import tpu as pltpu
from jax.experimental.pallas import tpu_sc as plsc

N_GRID = 1000000
N_OFFSET = 3
N_POINTS = 4194304
NP_TAB = N_GRID + 2 * N_OFFSET
NWIN = 2 * N_OFFSET + 1
N_ROWS = 500000

H = np.float32(1.0 / (N_GRID - 1))
FACTOR = np.float32(1.0 / np.sqrt(np.pi))

NC, NS, LANES = 2, 16, 16
NW = NC * NS

ROWS_PAD = 500224
ROWS_PER_TILE = ROWS_PAD // NW
PCH = 3908
N_PCH = ROWS_PER_TILE // PCH
SRC_PAD = 2 * ROWS_PAD + 16

PTS_PER_W = N_POINTS // NW
CHUNK = 1024
N_CHUNKS = PTS_PER_W // CHUNK
GROUPS = CHUNK // LANES
DMA_SLICE = 128
N_DMA = CHUNK // DMA_SLICE

_PARAMS = pltpu.CompilerParams(
    needs_layout_passes=False, use_tc_tiling_on_sc=False
)


def _prep_body(xg_hbm, u_hbm, tab_hbm, xgb, ub, tb):
    c = lax.axis_index("c")
    s = lax.axis_index("s")
    wid = s * NC + c
    tile_row0 = wid * ROWS_PER_TILE

    lane = lax.iota(jnp.int32, LANES)
    lowmask = lane < 8
    shdn8 = jnp.where(lane >= 8, lane - 8, 0)

    def chunk_body(pc, carry):
        row0 = tile_row0 + pc * PCH
        src0 = 2 * row0
        pltpu.sync_copy(xg_hbm.at[pl.ds(src0, 2 * PCH + 16)], xgb)
        pltpu.sync_copy(u_hbm.at[pl.ds(src0, 2 * PCH + 16)], ub)

        def row_body(rh, carry2):
            for t in range(2):
                rr = 2 * rh + t
                a = xgb[pl.ds(2 * rr, LANES)]
                b = ub[pl.ds(2 * rr, LANES)]
                bs = lax.gather(
                    b, shdn8[:, None],
                    lax.GatherDimensionNumbers(
                        offset_dims=(), collapsed_slice_dims=(0,),
                        start_index_map=(0,)),
                    (1,), mode=lax.GatherScatterMode.PROMISE_IN_BOUNDS,
                )
                tb[pl.ds(pl.multiple_of(rr * 16, 16), LANES)] = jnp.where(
                    lowmask, a, bs
                )
            return carry2

        lax.fori_loop(0, PCH // 2, row_body, 0)
        pltpu.sync_copy(tb, tab_hbm.at[pl.ds(row0 * 16, PCH * 16)])
        return carry

    lax.fori_loop(0, N_PCH, chunk_body, 0)


def _sc_body(x_hbm, tab_hbm, out_hbm, xbuf, rbuf, rows, ybuf, semx, semg, semy):
    c = lax.axis_index("c")
    s = lax.axis_index("s")
    wid = s * NC + c
    base = wid * PTS_PER_W

    lane = lax.iota(jnp.int32, LANES)

    def fire_x(ci, sl):
        pltpu.async_copy(
            x_hbm.at[pl.ds(base + ci * CHUNK, CHUNK)], xbuf.at[sl], semx
        )

    def wait_x(sl):
        pltpu.make_async_copy(
            x_hbm.at[pl.ds(base, CHUNK)], xbuf.at[sl], semx
        ).wait()

    def idx_pass(sl):
        def idx_body(rh, carry2):
            for t in range(2):
                r = 2 * rh + t
                ssl = pl.ds(pl.multiple_of(r * LANES, LANES), LANES)
                v = xbuf[sl, ssl]
                q = (v / H).astype(jnp.int32)
                rbuf[sl, ssl] = lax.shift_right_arithmetic(q, 1)
            return carry2

        lax.fori_loop(0, GROUPS // 2, idx_body, 0)

    def fire_g(sl):
        for j in range(N_DMA):
            jsl = pl.ds(j * DMA_SLICE, DMA_SLICE)
            pltpu.async_copy(
                tab_hbm.at[rbuf.at[sl, jsl]], rows.at[sl, jsl], semg
            )

    def wait_g(sl):
        for j in range(N_DMA):
            jsl = pl.ds(j * DMA_SLICE, DMA_SLICE)
            pltpu.make_async_copy(
                tab_hbm.at[rbuf.at[sl, jsl]], rows.at[sl, jsl], semg
            ).wait()

    def compute(sl):
        rows_sl = rows.at[sl]

        def cmp_body(rh, carry2):
            for t in range(2):
                r = 2 * rh + t
                ssl = pl.ds(pl.multiple_of(r * LANES, LANES), LANES)
                v = xbuf[sl, ssl]
                q = (v / H).astype(jnp.int32)
                cb = jnp.bitwise_and(q, 1)
                p = r * LANES + lane
                y1 = jnp.zeros((LANES,), jnp.float32)
                y = jnp.zeros((LANES,), jnp.float32)
                for i in range(NWIN):
                    col = cb + i
                    xg = plsc.load_gather(rows_sl, [p, col])
                    uu = plsc.load_gather(rows_sl, [p, col + 8])
                    d = (v - xg) / H
                    e = jnp.exp(-(d * d))
                    y1 = y1 + e
                    y = y + e * uu
                ybuf[sl, ssl] = y * FACTOR / y1
            return carry2

        lax.fori_loop(0, GROUPS // 2, cmp_body, 0)

    def fire_y(ci, sl):
        pltpu.async_copy(
            ybuf.at[sl], out_hbm.at[pl.ds(base + ci * CHUNK, CHUNK)], semy
        )

    def wait_y(sl):
        pltpu.make_async_copy(
            ybuf.at[sl], out_hbm.at[pl.ds(base, CHUNK)], semy
        ).wait()

    fire_x(0, 0)
    wait_x(0)
    idx_pass(0)
    fire_g(0)
    fire_x(1, 1)

    def pair_body(k, carry):
        c0 = 2 * k
        last = k == (N_CHUNKS // 2 - 1)
        wait_x(1)
        idx_pass(1)
        fire_g(1)
        wait_g(0)

        @pl.when(k > 0)
        def _():
            wait_y(0)

        compute(0)
        fire_y(c0, 0)

        @pl.when(jnp.logical_not(last))
        def _():
            fire_x(c0 + 2, 0)

        @pl.when(jnp.logical_not(last))
        def _():
            wait_x(0)
            idx_pass(0)
            fire_g(0)

        wait_g(1)

        @pl.when(k > 0)
        def _():
            wait_y(1)

        compute(1)
        fire_y(c0 + 1, 1)

        @pl.when(jnp.logical_not(last))
        def _():
            fire_x(c0 + 3, 1)

        return carry

    lax.fori_loop(0, N_CHUNKS // 2, pair_body, 0)
    wait_y(0)
    wait_y(1)


@jax.jit
def kernel(x, u):
    dx = 1.0 / (N_GRID - 1)
    xgrid = jnp.linspace(
        -dx * N_OFFSET, 1.0 + dx * N_OFFSET, NP_TAB, dtype=jnp.float32
    )
    zpad = jnp.zeros((SRC_PAD - NP_TAB,), jnp.float32)
    xgp = jnp.concatenate([xgrid, zpad])
    up = jnp.concatenate([u, zpad])

    mesh = plsc.VectorSubcoreMesh(
        core_axis_name="c", subcore_axis_name="s", num_cores=NC, num_subcores=NS
    )

    prep = pl.kernel(
        _prep_body,
        out_type=jax.ShapeDtypeStruct((ROWS_PAD * 16,), jnp.float32),
        mesh=mesh,
        compiler_params=_PARAMS,
        scratch_types=[
            pltpu.VMEM((2 * PCH + 16,), jnp.float32),
            pltpu.VMEM((2 * PCH + 16,), jnp.float32),
            pltpu.VMEM((PCH * 16,), jnp.float32),
        ],
    )
    tab = prep(xgp, up).reshape(ROWS_PAD, 16)

    run = pl.kernel(
        _sc_body,
        out_type=jax.ShapeDtypeStruct((N_POINTS,), jnp.float32),
        mesh=mesh,
        compiler_params=_PARAMS,
        scratch_types=[
            pltpu.VMEM((2, CHUNK), jnp.float32),
            pltpu.VMEM((2, CHUNK), jnp.int32),
            pltpu.VMEM((2, CHUNK, 16), jnp.float32),
            pltpu.VMEM((2, CHUNK), jnp.float32),
            pltpu.SemaphoreType.DMA,
            pltpu.SemaphoreType.DMA,
            pltpu.SemaphoreType.DMA,
        ],
    )
    return run(x, tab)


# speedup vs baseline: 1.0608x; 1.0608x over previous
import jax
import jax.numpy as jnp
import numpy as np
from jax import lax
from jax.experimental import pallas as pl
from jax.experimental.pallas import tpu as pltpu
from jax.experimental.pallas import tpu_sc as plsc

N_GRID = 1000000
N_OFFSET = 3
N_POINTS = 4194304
NP_TAB = N_GRID + 2 * N_OFFSET
NWIN = 2 * N_OFFSET + 1
N_ROWS = 500000

H = np.float32(1.0 / (N_GRID - 1))
FACTOR = np.float32(1.0 / np.sqrt(np.pi))

NC, NS, LANES = 2, 16, 16
NW = NC * NS

ROWS_PAD = 500224
ROWS_PER_TILE = ROWS_PAD // NW
PCH = 3908
N_PCH = ROWS_PER_TILE // PCH
SRC_PAD = 2 * ROWS_PAD + 16

PTS_PER_W = N_POINTS // NW
CHUNK = 2048
N_CHUNKS = PTS_PER_W // CHUNK
GROUPS = CHUNK // LANES
DMA_SLICE = 128
N_DMA = CHUNK // DMA_SLICE

_PARAMS = pltpu.CompilerParams(
    needs_layout_passes=False, use_tc_tiling_on_sc=False
)


def _prep_body(xg_hbm, u_hbm, tab_hbm, xgb, ub, tb):
    c = lax.axis_index("c")
    s = lax.axis_index("s")
    wid = s * NC + c
    tile_row0 = wid * ROWS_PER_TILE

    lane = lax.iota(jnp.int32, LANES)
    lowmask = lane < 8
    shdn8 = jnp.where(lane >= 8, lane - 8, 0)

    def chunk_body(pc, carry):
        row0 = tile_row0 + pc * PCH
        src0 = 2 * row0
        pltpu.sync_copy(xg_hbm.at[pl.ds(src0, 2 * PCH + 16)], xgb)
        pltpu.sync_copy(u_hbm.at[pl.ds(src0, 2 * PCH + 16)], ub)

        def row_body(rr, carry2):
            a = xgb[pl.ds(2 * rr, LANES)]
            b = ub[pl.ds(2 * rr, LANES)]
            bs = lax.gather(
                b, shdn8[:, None],
                lax.GatherDimensionNumbers(
                    offset_dims=(), collapsed_slice_dims=(0,),
                    start_index_map=(0,)),
                (1,), mode=lax.GatherScatterMode.PROMISE_IN_BOUNDS,
            )
            tb[pl.ds(pl.multiple_of(rr * 16, 16), LANES)] = jnp.where(
                lowmask, a, bs
            )
            return carry2

        lax.fori_loop(0, PCH, row_body, 0)
        pltpu.sync_copy(tb, tab_hbm.at[pl.ds(row0 * 16, PCH * 16)])
        return carry

    lax.fori_loop(0, N_PCH, chunk_body, 0)


def _sc_body(x_hbm, tab_hbm, out_hbm, xbuf, rbuf, rows, ybuf, semx, semg, semy):
    c = lax.axis_index("c")
    s = lax.axis_index("s")
    wid = s * NC + c
    base = wid * PTS_PER_W

    lane = lax.iota(jnp.int32, LANES)

    def fire_x(ci, sl):
        pltpu.async_copy(
            x_hbm.at[pl.ds(base + ci * CHUNK, CHUNK)], xbuf.at[sl], semx
        )

    def wait_x(sl):
        pltpu.make_async_copy(
            x_hbm.at[pl.ds(base, CHUNK)], xbuf.at[sl], semx
        ).wait()

    def idx_pass(sl):
        def idx_body(r, carry2):
            ssl = pl.ds(pl.multiple_of(r * LANES, LANES), LANES)
            v = xbuf[sl, ssl]
            q = (v / H).astype(jnp.int32)
            rbuf[sl, ssl] = lax.shift_right_arithmetic(q, 1)
            return carry2

        lax.fori_loop(0, GROUPS, idx_body, 0)

    def fire_g(sl):
        for j in range(N_DMA):
            jsl = pl.ds(j * DMA_SLICE, DMA_SLICE)
            pltpu.async_copy(
                tab_hbm.at[rbuf.at[sl, jsl]], rows.at[sl, jsl], semg
            )

    def wait_g(sl):
        for j in range(N_DMA):
            jsl = pl.ds(j * DMA_SLICE, DMA_SLICE)
            pltpu.make_async_copy(
                tab_hbm.at[rbuf.at[sl, jsl]], rows.at[sl, jsl], semg
            ).wait()

    def compute(sl):
        def cmp_body(r, carry2):
            ssl = pl.ds(pl.multiple_of(r * LANES, LANES), LANES)
            v = xbuf[sl, ssl]
            q = (v / H).astype(jnp.int32)
            cb = jnp.bitwise_and(q, 1)
            p = r * LANES + lane
            y1 = jnp.zeros((LANES,), jnp.float32)
            y = jnp.zeros((LANES,), jnp.float32)
            rows_sl = rows.at[sl]
            for i in range(NWIN):
                col = cb + i
                xg = plsc.load_gather(rows_sl, [p, col])
                uu = plsc.load_gather(rows_sl, [p, col + 8])
                d = (v - xg) / H
                e = jnp.exp(-(d * d))
                y1 = y1 + e
                y = y + e * uu
            ybuf[sl, ssl] = y * FACTOR / y1
            return carry2

        lax.fori_loop(0, GROUPS, cmp_body, 0)

    def fire_y(ci, sl):
        pltpu.async_copy(
            ybuf.at[sl], out_hbm.at[pl.ds(base + ci * CHUNK, CHUNK)], semy
        )

    def wait_y(sl):
        pltpu.make_async_copy(
            ybuf.at[sl], out_hbm.at[pl.ds(base, CHUNK)], semy
        ).wait()

    fire_x(0, 0)
    wait_x(0)
    idx_pass(0)
    fire_g(0)
    fire_x(1, 1)

    def pair_body(k, carry):
        c0 = 2 * k
        last = k == (N_CHUNKS // 2 - 1)
        wait_x(1)
        idx_pass(1)
        fire_g(1)
        wait_g(0)

        @pl.when(k > 0)
        def _():
            wait_y(0)

        compute(0)
        fire_y(c0, 0)

        @pl.when(jnp.logical_not(last))
        def _():
            fire_x(c0 + 2, 0)

        @pl.when(jnp.logical_not(last))
        def _():
            wait_x(0)
            idx_pass(0)
            fire_g(0)

        wait_g(1)

        @pl.when(k > 0)
        def _():
            wait_y(1)

        compute(1)
        fire_y(c0 + 1, 1)

        @pl.when(jnp.logical_not(last))
        def _():
            fire_x(c0 + 3, 1)

        return carry

    lax.fori_loop(0, N_CHUNKS // 2, pair_body, 0)
    wait_y(0)
    wait_y(1)


@jax.jit
def kernel(x, u):
    dx = 1.0 / (N_GRID - 1)
    xgrid = jnp.linspace(
        -dx * N_OFFSET, 1.0 + dx * N_OFFSET, NP_TAB, dtype=jnp.float32
    )
    zpad = jnp.zeros((SRC_PAD - NP_TAB,), jnp.float32)
    xgp = jnp.concatenate([xgrid, zpad])
    up = jnp.concatenate([u, zpad])

    mesh = plsc.VectorSubcoreMesh(
        core_axis_name="c", subcore_axis_name="s", num_cores=NC, num_subcores=NS
    )

    prep = pl.kernel(
        _prep_body,
        out_type=jax.ShapeDtypeStruct((ROWS_PAD * 16,), jnp.float32),
        mesh=mesh,
        compiler_params=_PARAMS,
        scratch_types=[
            pltpu.VMEM((2 * PCH + 16,), jnp.float32),
            pltpu.VMEM((2 * PCH + 16,), jnp.float32),
            pltpu.VMEM((PCH * 16,), jnp.float32),
        ],
    )
    tab = prep(xgp, up).reshape(ROWS_PAD, 16)

    run = pl.kernel(
        _sc_body,
        out_type=jax.ShapeDtypeStruct((N_POINTS,), jnp.float32),
        mesh=mesh,
        compiler_params=_PARAMS,
        scratch_types=[
            pltpu.VMEM((2, CHUNK), jnp.float32),
            pltpu.VMEM((2, CHUNK), jnp.int32),
            pltpu.VMEM((2, CHUNK, 16), jnp.float32),
            pltpu.VMEM((2, CHUNK), jnp.float32),
            pltpu.SemaphoreType.DMA,
            pltpu.SemaphoreType.DMA,
            pltpu.SemaphoreType.DMA,
        ],
    )
    return run(x, tab)
